# baseline (speedup 1.0000x reference)
import jax
import jax.numpy as jnp
from jax.experimental import pallas as pl
from jax.experimental.pallas import tpu as pltpu

_E = 8
_H = 768
_C = 1024
_K = 4
_NTOK = 32768
_NCHUNK = _NTOK // _C
_NGROUP = _NCHUNK // _K


def _compute_chunk(x, w, b, lstage, wstage, istage):
    logits = jnp.dot(x, w, preferred_element_type=jnp.float32) + b
    lstage[...] = logits
    lt = logits.T
    m = jnp.max(lt, axis=0, keepdims=True)
    ex = jnp.exp(lt - m)
    s = jnp.sum(ex, axis=0, keepdims=True)
    iota = jax.lax.broadcasted_iota(jnp.int32, lt.shape, 0)
    sentinel = jnp.int32(_E)
    idx1 = jnp.min(jnp.where(lt == m, iota, sentinel), axis=0, keepdims=True)
    masked = jnp.where(iota == idx1, -jnp.inf, lt)
    m2 = jnp.max(masked, axis=0, keepdims=True)
    idx2 = jnp.min(jnp.where(masked == m2, iota, sentinel), axis=0, keepdims=True)
    w1 = jnp.ones_like(s) / s
    w2 = jnp.exp(m2 - m) / s
    wstage[...] = jnp.concatenate([w1, w2], axis=0).T
    istage[...] = jnp.concatenate([idx1, idx2], axis=0).T


def _body(x_ref, w_ref, b_ref, logits_ref, weights_ref, index_ref,
          xbuf, lstage, wstage, istage,
          in_sem, l_sem, w_sem, i_sem):
    w = w_ref[...]
    b = b_ref[...]

    def issue_in(slot, chunk):
        pltpu.make_async_copy(
            x_ref.at[pl.ds(chunk * _C, _C), :], xbuf.at[slot], in_sem.at[slot]
        ).start()

    for k in range(_K):
        issue_in(k, k)

    def group_body(g, carry):
        p = jax.lax.rem(g, 2)
        pnext = 1 - p

        @pl.when(g + 1 < _NGROUP)
        def _():
            for k in range(_K):
                issue_in(pnext * _K + k, (g + 1) * _K + k)

        for k in range(_K):
            slot = p * _K + k
            chunk = g * _K + k
            off = chunk * _C
            pltpu.make_async_copy(
                x_ref.at[pl.ds(off, _C), :], xbuf.at[slot], in_sem.at[slot]
            ).wait()

            @pl.when(g >= 2)
            def _():
                pltpu.make_async_copy(
                    lstage.at[slot], logits_ref.at[pl.ds(off, _C), :], l_sem.at[slot]
                ).wait()
                pltpu.make_async_copy(
                    wstage.at[slot], weights_ref.at[pl.ds(off, _C), :], w_sem.at[slot]
                ).wait()
                pltpu.make_async_copy(
                    istage.at[slot], index_ref.at[pl.ds(off, _C), :], i_sem.at[slot]
                ).wait()

            _compute_chunk(xbuf[slot], w, b,
                           lstage.at[slot], wstage.at[slot], istage.at[slot])

            pltpu.make_async_copy(
                lstage.at[slot], logits_ref.at[pl.ds(off, _C), :], l_sem.at[slot]
            ).start()
            pltpu.make_async_copy(
                wstage.at[slot], weights_ref.at[pl.ds(off, _C), :], w_sem.at[slot]
            ).start()
            pltpu.make_async_copy(
                istage.at[slot], index_ref.at[pl.ds(off, _C), :], i_sem.at[slot]
            ).start()
        return carry

    jax.lax.fori_loop(0, _NGROUP, group_body, 0)

    for g in (_NGROUP - 2, _NGROUP - 1):
        p = g % 2
        for k in range(_K):
            slot = p * _K + k
            off = (g * _K + k) * _C
            pltpu.make_async_copy(
                lstage.at[slot], logits_ref.at[pl.ds(off, _C), :], l_sem.at[slot]
            ).wait()
            pltpu.make_async_copy(
                wstage.at[slot], weights_ref.at[pl.ds(off, _C), :], w_sem.at[slot]
            ).wait()
            pltpu.make_async_copy(
                istage.at[slot], index_ref.at[pl.ds(off, _C), :], i_sem.at[slot]
            ).wait()


def kernel(hidden_states, W, b):
    batch, seq, hidden = hidden_states.shape
    n_tokens = batch * seq
    x = hidden_states.reshape(n_tokens, hidden)
    wt = W.T
    b2 = b.reshape(1, _E)

    logits, weights, index = pl.pallas_call(
        _body,
        in_specs=[
            pl.BlockSpec(memory_space=pl.ANY),
            pl.BlockSpec(memory_space=pltpu.VMEM),
            pl.BlockSpec(memory_space=pltpu.VMEM),
        ],
        out_specs=[
            pl.BlockSpec(memory_space=pl.ANY),
            pl.BlockSpec(memory_space=pl.ANY),
            pl.BlockSpec(memory_space=pl.ANY),
        ],
        out_shape=[
            jax.ShapeDtypeStruct((n_tokens, _E), jnp.float32),
            jax.ShapeDtypeStruct((n_tokens, 2), jnp.float32),
            jax.ShapeDtypeStruct((n_tokens, 2), jnp.int32),
        ],
        scratch_shapes=[
            pltpu.VMEM((2 * _K, _C, _H), jnp.float32),
            pltpu.VMEM((2 * _K, _C, _E), jnp.float32),
            pltpu.VMEM((2 * _K, _C, 2), jnp.float32),
            pltpu.VMEM((2 * _K, _C, 2), jnp.int32),
            pltpu.SemaphoreType.DMA((2 * _K,)),
            pltpu.SemaphoreType.DMA((2 * _K,)),
            pltpu.SemaphoreType.DMA((2 * _K,)),
            pltpu.SemaphoreType.DMA((2 * _K,)),
        ],
    )(x, wt, b2)

    return (
        index.reshape(-1),
        weights.reshape(batch, seq, 2),
        logits.reshape(batch, seq, _E),
    )


# speedup vs baseline: 1.0367x; 1.0367x over previous
import jax
import jax.numpy as jnp
from jax.experimental import pallas as pl
from jax.experimental.pallas import tpu as pltpu

_NUM_EXPERTS = 8
_HIDDEN = 768
_BLOCK = 1024
_NSLICE = 4


def _route_slice(x, w, b, j, logits_ref, weights_ref, index_ref):
    logits = jnp.dot(x, w, preferred_element_type=jnp.float32) + b
    logits_ref[pl.ds(j * _BLOCK, _BLOCK), :] = logits

    lt = logits.T

    m = jnp.max(lt, axis=0, keepdims=True)
    ex = jnp.exp(lt - m)
    s = jnp.sum(ex, axis=0, keepdims=True)

    iota = jax.lax.broadcasted_iota(jnp.int32, lt.shape, 0)
    sentinel = jnp.int32(_NUM_EXPERTS)
    idx1 = jnp.min(jnp.where(lt == m, iota, sentinel), axis=0, keepdims=True)
    masked = jnp.where(iota == idx1, -jnp.inf, lt)
    m2 = jnp.max(masked, axis=0, keepdims=True)
    idx2 = jnp.min(jnp.where(masked == m2, iota, sentinel), axis=0, keepdims=True)

    w1 = jnp.ones_like(s) / s
    w2 = jnp.exp(m2 - m) / s
    weights_ref[pl.ds(j * _BLOCK, _BLOCK), :] = jnp.concatenate([w1, w2], axis=0).T
    index_ref[pl.ds(j * _BLOCK, _BLOCK), :] = jnp.concatenate([idx1, idx2], axis=0).T


def _router_block(*refs):
    s = _NSLICE
    xs = refs[:s]
    w = refs[s][...]
    b = refs[s + 1][...]
    logits_ref, weights_ref, index_ref = refs[s + 2:s + 5]
    for j in range(s):
        _route_slice(xs[j][...], w, b, j, logits_ref, weights_ref, index_ref)


def kernel(hidden_states, W, b):
    batch, seq, hidden = hidden_states.shape
    n_tokens = batch * seq
    x = hidden_states.reshape(n_tokens, hidden)
    wt = W.T
    b2 = b.reshape(1, _NUM_EXPERTS)

    s = _NSLICE
    step = _BLOCK * s
    grid = (n_tokens // step,)

    def slice_map(j):
        return lambda i: (i * s + j, 0)

    in_specs = [pl.BlockSpec((_BLOCK, hidden), slice_map(j)) for j in range(s)]
    in_specs += [
        pl.BlockSpec((hidden, _NUM_EXPERTS), lambda i: (0, 0)),
        pl.BlockSpec((1, _NUM_EXPERTS), lambda i: (0, 0)),
    ]
    out_specs = [
        pl.BlockSpec((step, _NUM_EXPERTS), lambda i: (i, 0)),
        pl.BlockSpec((step, 2), lambda i: (i, 0)),
        pl.BlockSpec((step, 2), lambda i: (i, 0)),
    ]
    out_shape = [
        jax.ShapeDtypeStruct((n_tokens, _NUM_EXPERTS), jnp.float32),
        jax.ShapeDtypeStruct((n_tokens, 2), jnp.float32),
        jax.ShapeDtypeStruct((n_tokens, 2), jnp.int32),
    ]
    logits, weights, index = pl.pallas_call(
        _router_block,
        grid=grid,
        in_specs=in_specs,
        out_specs=out_specs,
        out_shape=out_shape,
        compiler_params=pltpu.CompilerParams(
            dimension_semantics=("arbitrary",),
            allow_input_fusion=[True] * (_NSLICE + 2),
        ),
    )(*([x] * s), wt, b2)

    return (
        index.reshape(-1),
        weights.reshape(batch, seq, 2),
        logits.reshape(batch, seq, _NUM_EXPERTS),
    )


# speedup vs baseline: 1.0370x; 1.0002x over previous
import jax
import jax.numpy as jnp
from jax.experimental import pallas as pl
from jax.experimental.pallas import tpu as pltpu

_NUM_EXPERTS = 8
_HIDDEN = 768
_BLOCK = 512
_NSLICE = 8


def _route_slice(x, w, b, j, logits_ref, weights_ref, index_ref):
    logits = jnp.dot(x, w, preferred_element_type=jnp.float32) + b
    logits_ref[pl.ds(j * _BLOCK, _BLOCK), :] = logits

    lt = logits.T

    m = jnp.max(lt, axis=0, keepdims=True)
    ex = jnp.exp(lt - m)
    s = jnp.sum(ex, axis=0, keepdims=True)

    iota = jax.lax.broadcasted_iota(jnp.int32, lt.shape, 0)
    sentinel = jnp.int32(_NUM_EXPERTS)
    idx1 = jnp.min(jnp.where(lt == m, iota, sentinel), axis=0, keepdims=True)
    masked = jnp.where(iota == idx1, -jnp.inf, lt)
    m2 = jnp.max(masked, axis=0, keepdims=True)
    idx2 = jnp.min(jnp.where(masked == m2, iota, sentinel), axis=0, keepdims=True)

    w1 = jnp.ones_like(s) / s
    w2 = jnp.exp(m2 - m) / s
    weights_ref[pl.ds(j * _BLOCK, _BLOCK), :] = jnp.concatenate([w1, w2], axis=0).T
    index_ref[pl.ds(j * _BLOCK, _BLOCK), :] = jnp.concatenate([idx1, idx2], axis=0).T


def _router_block(*refs):
    s = _NSLICE
    xs = refs[:s]
    w = refs[s][...]
    b = refs[s + 1][...]
    logits_ref, weights_ref, index_ref = refs[s + 2:s + 5]
    for j in range(s):
        _route_slice(xs[j][...], w, b, j, logits_ref, weights_ref, index_ref)


def kernel(hidden_states, W, b):
    batch, seq, hidden = hidden_states.shape
    n_tokens = batch * seq
    x = hidden_states.reshape(n_tokens, hidden)
    wt = W.T
    b2 = b.reshape(1, _NUM_EXPERTS)

    s = _NSLICE
    step = _BLOCK * s
    grid = (n_tokens // step,)

    def slice_map(j):
        return lambda i: (i * s + j, 0)

    in_specs = [pl.BlockSpec((_BLOCK, hidden), slice_map(j)) for j in range(s)]
    in_specs += [
        pl.BlockSpec((hidden, _NUM_EXPERTS), lambda i: (0, 0)),
        pl.BlockSpec((1, _NUM_EXPERTS), lambda i: (0, 0)),
    ]
    out_specs = [
        pl.BlockSpec((step, _NUM_EXPERTS), lambda i: (i, 0)),
        pl.BlockSpec((step, 2), lambda i: (i, 0)),
        pl.BlockSpec((step, 2), lambda i: (i, 0)),
    ]
    out_shape = [
        jax.ShapeDtypeStruct((n_tokens, _NUM_EXPERTS), jnp.float32),
        jax.ShapeDtypeStruct((n_tokens, 2), jnp.float32),
        jax.ShapeDtypeStruct((n_tokens, 2), jnp.int32),
    ]
    logits, weights, index = pl.pallas_call(
        _router_block,
        grid=grid,
        in_specs=in_specs,
        out_specs=out_specs,
        out_shape=out_shape,
        compiler_params=pltpu.CompilerParams(
            dimension_semantics=("arbitrary",),
            allow_input_fusion=[True] * (_NSLICE + 2),
        ),
    )(*([x] * s), wt, b2)

    return (
        index.reshape(-1),
        weights.reshape(batch, seq, 2),
        logits.reshape(batch, seq, _NUM_EXPERTS),
    )


# speedup vs baseline: 1.0392x; 1.0021x over previous
import jax
import jax.numpy as jnp
from jax.experimental import pallas as pl
from jax.experimental.pallas import tpu as pltpu

_NUM_EXPERTS = 8
_HIDDEN = 768
_BLOCK = 1024
_NSLICE = 4


def _route_slice(x, w, b, j, logits_ref, weights_ref, index_ref):
    logits = jnp.dot(x, w, preferred_element_type=jnp.float32) + b
    logits_ref[pl.ds(j * _BLOCK, _BLOCK), :] = logits

    lt = logits.T

    m = jnp.max(lt, axis=0, keepdims=True)
    ex = jnp.exp(lt - m)
    s = jnp.sum(ex, axis=0, keepdims=True)

    iota = jax.lax.broadcasted_iota(jnp.int32, lt.shape, 0)
    sentinel = jnp.int32(_NUM_EXPERTS)
    idx1 = jnp.min(jnp.where(lt == m, iota, sentinel), axis=0, keepdims=True)
    masked = jnp.where(iota == idx1, -jnp.inf, lt)
    m2 = jnp.max(masked, axis=0, keepdims=True)
    idx2 = jnp.min(jnp.where(masked == m2, iota, sentinel), axis=0, keepdims=True)

    w1 = jnp.ones_like(s) / s
    w2 = jnp.exp(m2 - m) / s
    weights_ref[pl.ds(j * _BLOCK, _BLOCK), :] = jnp.concatenate([w1, w2], axis=0).T
    index_ref[pl.ds(j * _BLOCK, _BLOCK), :] = jnp.concatenate([idx1, idx2], axis=0).T


def _router_block(*refs):
    s = _NSLICE
    xs = refs[:s]
    w = refs[s][...]
    b = refs[s + 1][...]
    logits_ref, weights_ref, index_ref = refs[s + 2:s + 5]
    for j in range(s):
        _route_slice(xs[j][...], w, b, j, logits_ref, weights_ref, index_ref)


def kernel(hidden_states, W, b):
    batch, seq, hidden = hidden_states.shape
    n_tokens = batch * seq
    x = hidden_states.reshape(n_tokens, hidden)
    wt = W.T
    b2 = b.reshape(1, _NUM_EXPERTS)

    s = _NSLICE
    step = _BLOCK * s
    grid = (n_tokens // step,)

    def slice_map(j):
        return lambda i: (i * s + j, 0)

    in_specs = [pl.BlockSpec((_BLOCK, hidden), slice_map(j)) for j in range(s)]
    in_specs += [
        pl.BlockSpec((hidden, _NUM_EXPERTS), lambda i: (0, 0)),
        pl.BlockSpec((1, _NUM_EXPERTS), lambda i: (0, 0)),
    ]
    out_specs = [
        pl.BlockSpec((step, _NUM_EXPERTS), lambda i: (i, 0)),
        pl.BlockSpec((step, 2), lambda i: (i, 0)),
        pl.BlockSpec((step, 2), lambda i: (i, 0)),
    ]
    out_shape = [
        jax.ShapeDtypeStruct((n_tokens, _NUM_EXPERTS), jnp.float32),
        jax.ShapeDtypeStruct((n_tokens, 2), jnp.float32),
        jax.ShapeDtypeStruct((n_tokens, 2), jnp.int32),
    ]
    logits, weights, index = pl.pallas_call(
        _router_block,
        grid=grid,
        in_specs=in_specs,
        out_specs=out_specs,
        out_shape=out_shape,
        compiler_params=pltpu.CompilerParams(
            dimension_semantics=("arbitrary",),
            allow_input_fusion=[True] * (_NSLICE + 2),
        ),
    )(*([x] * s), wt, b2)

    return (
        index.reshape(-1),
        weights.reshape(batch, seq, 2),
        logits.reshape(batch, seq, _NUM_EXPERTS),
    )
